# baseline (speedup 1.0000x reference)
import numpy as np
import jax
import jax.numpy as jnp
from jax import lax
from jax.experimental import pallas as pl

B = 32
L = 128
F = 32
D_OB = 4
D_MODEL = F * D_OB
D_PE = 16
D = D_MODEL + D_PE
H = 12
HD = D // H
D_FFN = 512
N_LAYERS = 2
C = L * D_OB

_TIMESCALES = np.asarray(float(L) ** np.linspace(0.0, 1.0, D_PE // 2),
                         dtype=np.float32)


def _prop_pe_body(xg_ref, rb_ref, w1v_ref, b1v_ref, w1s_ref, b1s_ref,
                  w2v_ref, b2v_ref, w2s_ref, b2s_ref, times_ref, ts_ref,
                  z_ref, pes_ref, pec_ref):
    s = jax.nn.relu(xg_ref[...] * rb_ref[...]).astype(jnp.bfloat16)
    y = (jax.nn.relu(jnp.dot(s, w1v_ref[...], preferred_element_type=jnp.float32)
                     + b1v_ref[...])
         + jnp.dot(s, w1s_ref[...], preferred_element_type=jnp.float32)
         + b1s_ref[...]).astype(jnp.bfloat16)
    z = (jax.nn.relu(jnp.dot(y, w2v_ref[...], preferred_element_type=jnp.float32)
                     + b2v_ref[...])
         + jnp.dot(y, w2s_ref[...], preferred_element_type=jnp.float32)
         + b2s_ref[...])
    z_ref[...] = z
    scaled = times_ref[...][:, :, None] / ts_ref[...][None, :, :]
    pes_ref[...] = jnp.sin(scaled)
    pec_ref[...] = jnp.cos(scaled)


def _ln(t, w, b):
    mu = jnp.mean(t, axis=-1, keepdims=True)
    var = jnp.mean((t - mu) ** 2, axis=-1, keepdims=True)
    return (t - mu) / jnp.sqrt(var + 1e-5) * w + b


def _tf_body(x_ref, neg_ref, wq_ref, wk_ref, wv_ref, bq_ref, bk_ref, bv_ref,
             wo_ref, bo_ref, w1_ref, b1_ref, w2_ref, b2_ref,
             n1w_ref, n1b_ref, n2w_ref, n2b_ref, out_ref):
    x = x_ref[0]
    neg = neg_ref[0]
    scale = 1.0 / float(np.sqrt(HD))
    col = lax.broadcasted_iota(jnp.int32, (1, D), 1)
    bf16 = jnp.bfloat16
    for l in range(N_LAYERS):
        x16 = x.astype(bf16)
        q = jnp.dot(x16, wq_ref[l], preferred_element_type=jnp.float32) + bq_ref[l]
        k = jnp.dot(x16, wk_ref[l], preferred_element_type=jnp.float32) + bk_ref[l]
        v = jnp.dot(x16, wv_ref[l], preferred_element_type=jnp.float32) + bv_ref[l]
        masks = [(col // HD == h).astype(jnp.float32) for h in range(H)]
        kms = jnp.concatenate([k * mh for mh in masks], axis=0).astype(bf16)
        vms = jnp.concatenate([v * mh for mh in masks], axis=0).astype(bf16)
        s = lax.dot_general(q.astype(bf16), kms, (((1,), (1,)), ((), ())),
                            preferred_element_type=jnp.float32)
        s = s * scale
        ps = []
        for h in range(H):
            sh = s[:, h * L:(h + 1) * L] + neg
            m = jnp.max(sh, axis=-1, keepdims=True)
            e = jnp.exp(sh - m)
            ps.append((e / jnp.sum(e, axis=-1, keepdims=True)).astype(bf16))
        p = jnp.concatenate(ps, axis=1)
        o = jnp.dot(p, vms, preferred_element_type=jnp.float32)
        a = jnp.dot(o.astype(bf16), wo_ref[l],
                    preferred_element_type=jnp.float32) + bo_ref[l]
        x = _ln(x + a, n1w_ref[l], n1b_ref[l])
        f = jnp.dot(
            jax.nn.relu(
                jnp.dot(x.astype(bf16), w1_ref[l],
                        preferred_element_type=jnp.float32)
                + b1_ref[l]).astype(bf16),
            w2_ref[l], preferred_element_type=jnp.float32) + b2_ref[l]
        x = _ln(x + f, n2w_ref[l], n2b_ref[l])
    out_ref[0] = x


def kernel(X, timestamps, lengths, R_u, op1_vw, op1_vb, op1_sw, op1_sb,
           op2_vw, op2_vb, op2_sw, op2_sb, in_proj_w, in_proj_b,
           out_proj_w, out_proj_b, lin1_w, lin1_b, lin2_w, lin2_b,
           norm1_w, norm1_b, norm2_w, norm2_b):
    f32 = jnp.float32

    xt = X.transpose(0, 2, 1).reshape(B * F, L)
    xg = jnp.broadcast_to(xt[:, :, None], (B * F, L, D_OB)).reshape(B * F, C)
    rb_pat = jnp.broadcast_to(R_u.reshape(F, D_OB)[:, None, :],
                              (F, L, D_OB)).reshape(F, C)
    rb = jnp.tile(rb_pat, (B, 1))
    times_t = timestamps.transpose(1, 0)

    z, pe_sin, pe_cos = pl.pallas_call(
        _prop_pe_body,
        out_shape=[
            jax.ShapeDtypeStruct((B * F, C), f32),
            jax.ShapeDtypeStruct((L, B, D_PE // 2), f32),
            jax.ShapeDtypeStruct((L, B, D_PE // 2), f32),
        ],
    )(xg, rb,
      op1_vw.T.astype(jnp.bfloat16), op1_vb.reshape(1, C),
      op1_sw.T.astype(jnp.bfloat16), op1_sb.reshape(1, C),
      op2_vw.T.astype(jnp.bfloat16), op2_vb.reshape(1, C),
      op2_sw.T.astype(jnp.bfloat16), op2_sb.reshape(1, C),
      times_t, jnp.asarray(_TIMESCALES).reshape(1, D_PE // 2))

    out_units = z.reshape(B, F, L, D_OB).transpose(2, 0, 1, 3).reshape(L, B, D_MODEL)
    pe = jnp.concatenate([pe_sin, pe_cos], axis=-1)
    x0 = jnp.concatenate([out_units, pe], axis=2).transpose(1, 0, 2)

    mask = jnp.arange(L)[None, :] >= lengths
    neg = jnp.where(mask, jnp.float32(-1e30), jnp.float32(0.0))
    neg3 = neg.reshape(B, 1, L)

    wq = in_proj_w[:, 0 * D:1 * D, :].transpose(0, 2, 1).astype(jnp.bfloat16)
    wk = in_proj_w[:, 1 * D:2 * D, :].transpose(0, 2, 1).astype(jnp.bfloat16)
    wv = in_proj_w[:, 2 * D:3 * D, :].transpose(0, 2, 1).astype(jnp.bfloat16)
    bq = in_proj_b[:, 0 * D:1 * D].reshape(N_LAYERS, 1, D)
    bk = in_proj_b[:, 1 * D:2 * D].reshape(N_LAYERS, 1, D)
    bv = in_proj_b[:, 2 * D:3 * D].reshape(N_LAYERS, 1, D)

    full = lambda shape: pl.BlockSpec(shape, lambda b: (0,) * len(shape))
    xout = pl.pallas_call(
        _tf_body,
        grid=(B,),
        in_specs=[
            pl.BlockSpec((1, L, D), lambda b: (b, 0, 0)),
            pl.BlockSpec((1, 1, L), lambda b: (b, 0, 0)),
            full((N_LAYERS, D, D)), full((N_LAYERS, D, D)), full((N_LAYERS, D, D)),
            full((N_LAYERS, 1, D)), full((N_LAYERS, 1, D)), full((N_LAYERS, 1, D)),
            full((N_LAYERS, D, D)), full((N_LAYERS, 1, D)),
            full((N_LAYERS, D, D_FFN)), full((N_LAYERS, 1, D_FFN)),
            full((N_LAYERS, D_FFN, D)), full((N_LAYERS, 1, D)),
            full((N_LAYERS, 1, D)), full((N_LAYERS, 1, D)),
            full((N_LAYERS, 1, D)), full((N_LAYERS, 1, D)),
        ],
        out_specs=pl.BlockSpec((1, L, D), lambda b: (b, 0, 0)),
        out_shape=jax.ShapeDtypeStruct((B, L, D), f32),
    )(x0, neg3, wq, wk, wv, bq, bk, bv,
      out_proj_w.transpose(0, 2, 1).astype(jnp.bfloat16),
      out_proj_b.reshape(N_LAYERS, 1, D),
      lin1_w.transpose(0, 2, 1).astype(jnp.bfloat16),
      lin1_b.reshape(N_LAYERS, 1, D_FFN),
      lin2_w.transpose(0, 2, 1).astype(jnp.bfloat16),
      lin2_b.reshape(N_LAYERS, 1, D),
      norm1_w.reshape(N_LAYERS, 1, D), norm1_b.reshape(N_LAYERS, 1, D),
      norm2_w.reshape(N_LAYERS, 1, D), norm2_b.reshape(N_LAYERS, 1, D))

    return xout.transpose(1, 0, 2), mask


# speedup vs baseline: 1.5921x; 1.5921x over previous
import numpy as np
import jax
import jax.numpy as jnp
from jax import lax
from jax.experimental import pallas as pl

B = 32
L = 128
F = 32
D_OB = 4
D_MODEL = F * D_OB
D_PE = 16
D = D_MODEL + D_PE
H = 12
HD = D // H
D_FFN = 512
N_LAYERS = 2
C = L * D_OB
BPS = 4

_TIMESCALES = np.asarray(float(L) ** np.linspace(0.0, 1.0, D_PE // 2),
                         dtype=np.float32)


def _prop_pe_body(xg_ref, rb_ref, w1v_ref, b1v_ref, w1s_ref, b1s_ref,
                  w2v_ref, b2v_ref, w2s_ref, b2s_ref, tr_ref, tsr_ref,
                  z_ref, pes_ref, pec_ref):
    s = jax.nn.relu(xg_ref[...] * rb_ref[...])
    y = (jax.nn.relu(jnp.dot(s, w1v_ref[...], preferred_element_type=jnp.float32)
                     + b1v_ref[...])
         + jnp.dot(s, w1s_ref[...], preferred_element_type=jnp.float32)
         + b1s_ref[...])
    z = (jax.nn.relu(jnp.dot(y, w2v_ref[...], preferred_element_type=jnp.float32)
                     + b2v_ref[...])
         + jnp.dot(y, w2s_ref[...], preferred_element_type=jnp.float32)
         + b2s_ref[...])
    z_ref[...] = z
    scaled = tr_ref[...] / tsr_ref[...]
    pes_ref[...] = jnp.sin(scaled)
    pec_ref[...] = jnp.cos(scaled)


def _ln(t, w, b):
    mu = jnp.mean(t, axis=-1, keepdims=True)
    var = jnp.mean((t - mu) ** 2, axis=-1, keepdims=True)
    return (t - mu) / jnp.sqrt(var + 1e-5) * w + b


def _tf_body(x_ref, neg_ref, wq_ref, wk_ref, wv_ref, bq_ref, bk_ref, bv_ref,
             wo_ref, bo_ref, w1_ref, b1_ref, w2_ref, b2_ref,
             n1w_ref, n1b_ref, n2w_ref, n2b_ref, out_ref):
    x = x_ref[...].reshape(BPS * L, D)
    scale = 1.0 / float(np.sqrt(HD))
    col = lax.broadcasted_iota(jnp.int32, (1, D), 1)
    masks = [(col // HD == h).astype(jnp.float32) for h in range(H)]
    for l in range(N_LAYERS):
        q = jnp.dot(x, wq_ref[l], preferred_element_type=jnp.float32) + bq_ref[l]
        k = jnp.dot(x, wk_ref[l], preferred_element_type=jnp.float32) + bk_ref[l]
        v = jnp.dot(x, wv_ref[l], preferred_element_type=jnp.float32) + bv_ref[l]
        os = []
        for j in range(BPS):
            qj = q[j * L:(j + 1) * L]
            kj = k[j * L:(j + 1) * L]
            vj = v[j * L:(j + 1) * L]
            neg = neg_ref[j]
            kms = jnp.concatenate([kj * mh for mh in masks], axis=0)
            vms = jnp.concatenate([vj * mh for mh in masks], axis=0)
            s = lax.dot_general(qj, kms, (((1,), (1,)), ((), ())),
                                preferred_element_type=jnp.float32)
            s = s * scale
            ps = []
            for h in range(H):
                sh = s[:, h * L:(h + 1) * L] + neg
                m = jnp.max(sh, axis=-1, keepdims=True)
                e = jnp.exp(sh - m)
                ps.append(e / jnp.sum(e, axis=-1, keepdims=True))
            p = jnp.concatenate(ps, axis=1)
            os.append(jnp.dot(p, vms, preferred_element_type=jnp.float32))
        o = jnp.concatenate(os, axis=0)
        a = jnp.dot(o, wo_ref[l], preferred_element_type=jnp.float32) + bo_ref[l]
        x = _ln(x + a, n1w_ref[l], n1b_ref[l])
        f = jnp.dot(
            jax.nn.relu(
                jnp.dot(x, w1_ref[l], preferred_element_type=jnp.float32)
                + b1_ref[l]),
            w2_ref[l], preferred_element_type=jnp.float32) + b2_ref[l]
        x = _ln(x + f, n2w_ref[l], n2b_ref[l])
    out_ref[...] = x.reshape(BPS, L, D)


def kernel(X, timestamps, lengths, R_u, op1_vw, op1_vb, op1_sw, op1_sb,
           op2_vw, op2_vb, op2_sw, op2_sb, in_proj_w, in_proj_b,
           out_proj_w, out_proj_b, lin1_w, lin1_b, lin2_w, lin2_b,
           norm1_w, norm1_b, norm2_w, norm2_b):
    f32 = jnp.float32

    xt = X.transpose(0, 2, 1).reshape(B * F, L)
    xg = jnp.broadcast_to(xt[:, :, None], (B * F, L, D_OB)).reshape(B * F, C)
    rb_pat = jnp.broadcast_to(R_u.reshape(F, D_OB)[:, None, :],
                              (F, L, D_OB)).reshape(F, C)
    rb = jnp.tile(rb_pat, (B, 1))
    times_rep = jnp.repeat(timestamps.transpose(1, 0), D_PE // 2, axis=1)
    ts_rep = jnp.tile(jnp.asarray(_TIMESCALES).reshape(1, D_PE // 2), (1, B))

    z, pe_sin, pe_cos = pl.pallas_call(
        _prop_pe_body,
        out_shape=[
            jax.ShapeDtypeStruct((B * F, C), f32),
            jax.ShapeDtypeStruct((L, B * (D_PE // 2)), f32),
            jax.ShapeDtypeStruct((L, B * (D_PE // 2)), f32),
        ],
    )(xg, rb,
      op1_vw.T, op1_vb.reshape(1, C), op1_sw.T, op1_sb.reshape(1, C),
      op2_vw.T, op2_vb.reshape(1, C), op2_sw.T, op2_sb.reshape(1, C),
      times_rep, ts_rep)

    out_units = z.reshape(B, F, L, D_OB).transpose(2, 0, 1, 3).reshape(L, B, D_MODEL)
    pe = jnp.concatenate([pe_sin.reshape(L, B, D_PE // 2),
                          pe_cos.reshape(L, B, D_PE // 2)], axis=-1)
    x0 = jnp.concatenate([out_units, pe], axis=2).transpose(1, 0, 2)

    mask = jnp.arange(L)[None, :] >= lengths
    neg = jnp.where(mask, jnp.float32(-1e30), jnp.float32(0.0))
    neg3 = neg.reshape(B, 1, L)

    wq = in_proj_w[:, 0 * D:1 * D, :].transpose(0, 2, 1)
    wk = in_proj_w[:, 1 * D:2 * D, :].transpose(0, 2, 1)
    wv = in_proj_w[:, 2 * D:3 * D, :].transpose(0, 2, 1)
    bq = in_proj_b[:, 0 * D:1 * D].reshape(N_LAYERS, 1, D)
    bk = in_proj_b[:, 1 * D:2 * D].reshape(N_LAYERS, 1, D)
    bv = in_proj_b[:, 2 * D:3 * D].reshape(N_LAYERS, 1, D)

    full = lambda shape: pl.BlockSpec(shape, lambda b: (0,) * len(shape))
    xout = pl.pallas_call(
        _tf_body,
        grid=(B // BPS,),
        in_specs=[
            pl.BlockSpec((BPS, L, D), lambda b: (b, 0, 0)),
            pl.BlockSpec((BPS, 1, L), lambda b: (b, 0, 0)),
            full((N_LAYERS, D, D)), full((N_LAYERS, D, D)), full((N_LAYERS, D, D)),
            full((N_LAYERS, 1, D)), full((N_LAYERS, 1, D)), full((N_LAYERS, 1, D)),
            full((N_LAYERS, D, D)), full((N_LAYERS, 1, D)),
            full((N_LAYERS, D, D_FFN)), full((N_LAYERS, 1, D_FFN)),
            full((N_LAYERS, D_FFN, D)), full((N_LAYERS, 1, D)),
            full((N_LAYERS, 1, D)), full((N_LAYERS, 1, D)),
            full((N_LAYERS, 1, D)), full((N_LAYERS, 1, D)),
        ],
        out_specs=pl.BlockSpec((BPS, L, D), lambda b: (b, 0, 0)),
        out_shape=jax.ShapeDtypeStruct((B, L, D), f32),
    )(x0, neg3, wq, wk, wv, bq, bk, bv,
      out_proj_w.transpose(0, 2, 1), out_proj_b.reshape(N_LAYERS, 1, D),
      lin1_w.transpose(0, 2, 1), lin1_b.reshape(N_LAYERS, 1, D_FFN),
      lin2_w.transpose(0, 2, 1), lin2_b.reshape(N_LAYERS, 1, D),
      norm1_w.reshape(N_LAYERS, 1, D), norm1_b.reshape(N_LAYERS, 1, D),
      norm2_w.reshape(N_LAYERS, 1, D), norm2_b.reshape(N_LAYERS, 1, D))

    return xout.transpose(1, 0, 2), mask


# speedup vs baseline: 1.8610x; 1.1689x over previous
import numpy as np
import jax
import jax.numpy as jnp
from jax import lax
from jax.experimental import pallas as pl

B = 32
L = 128
F = 32
D_OB = 4
D_MODEL = F * D_OB
D_PE = 16
D = D_MODEL + D_PE
H = 12
HD = D // H
D_FFN = 512
N_LAYERS = 2
C = L * D_OB
BPS = 8

_TIMESCALES = np.asarray(float(L) ** np.linspace(0.0, 1.0, D_PE // 2),
                         dtype=np.float32)

_NT = (((1,), (1,)), ((), ()))


def _prop_pe_body(xt_ref, rp_ref, w1v_ref, b1v_ref, w1s_ref, b1s_ref,
                  w2v_ref, b2v_ref, w2s_ref, b2s_ref, tr_ref, tsr_ref,
                  z_ref, pes_ref, pec_ref):
    e = (lax.broadcasted_iota(jnp.int32, (L, C), 1) // D_OB
         == lax.broadcasted_iota(jnp.int32, (L, C), 0)).astype(jnp.float32)
    xg = jnp.dot(xt_ref[...], e, preferred_element_type=jnp.float32)
    rb = jnp.broadcast_to(rp_ref[...][None], (B, F, C)).reshape(B * F, C)
    s = jax.nn.relu(xg * rb)
    y = (jax.nn.relu(lax.dot_general(s, w1v_ref[...], _NT,
                                     preferred_element_type=jnp.float32)
                     + b1v_ref[...])
         + lax.dot_general(s, w1s_ref[...], _NT,
                           preferred_element_type=jnp.float32)
         + b1s_ref[...])
    z = (jax.nn.relu(lax.dot_general(y, w2v_ref[...], _NT,
                                     preferred_element_type=jnp.float32)
                     + b2v_ref[...])
         + lax.dot_general(y, w2s_ref[...], _NT,
                           preferred_element_type=jnp.float32)
         + b2s_ref[...])
    z_ref[...] = z
    scaled = tr_ref[...] / tsr_ref[...]
    pes_ref[...] = jnp.sin(scaled)
    pec_ref[...] = jnp.cos(scaled)


def _ln(t, w, b):
    mu = jnp.mean(t, axis=-1, keepdims=True)
    var = jnp.mean((t - mu) ** 2, axis=-1, keepdims=True)
    return (t - mu) / jnp.sqrt(var + 1e-5) * w + b


def _tf_body(x_ref, neg_ref, wq_ref, wk_ref, wv_ref, bq_ref, bk_ref, bv_ref,
             wo_ref, bo_ref, w1_ref, b1_ref, w2_ref, b2_ref,
             n1w_ref, n1b_ref, n2w_ref, n2b_ref, out_ref):
    x = x_ref[...].reshape(BPS * L, D)
    scale = 1.0 / float(np.sqrt(HD))
    col = lax.broadcasted_iota(jnp.int32, (1, D), 1)
    masks = [(col // HD == h).astype(jnp.float32) for h in range(H)]
    for l in range(N_LAYERS):
        q = lax.dot_general(x, wq_ref[l], _NT,
                            preferred_element_type=jnp.float32) + bq_ref[l]
        k = lax.dot_general(x, wk_ref[l], _NT,
                            preferred_element_type=jnp.float32) + bk_ref[l]
        v = lax.dot_general(x, wv_ref[l], _NT,
                            preferred_element_type=jnp.float32) + bv_ref[l]
        os = []
        for j in range(BPS):
            qj = q[j * L:(j + 1) * L]
            kj = k[j * L:(j + 1) * L]
            vj = v[j * L:(j + 1) * L]
            neg = neg_ref[j]
            kms = jnp.concatenate([kj * mh for mh in masks], axis=0)
            vms = jnp.concatenate([vj * mh for mh in masks], axis=0)
            s = lax.dot_general(qj, kms, _NT,
                                preferred_element_type=jnp.float32)
            s = s * scale
            ps = []
            for h in range(H):
                sh = s[:, h * L:(h + 1) * L] + neg
                m = jnp.max(sh, axis=-1, keepdims=True)
                e = jnp.exp(sh - m)
                ps.append(e / jnp.sum(e, axis=-1, keepdims=True))
            p = jnp.concatenate(ps, axis=1)
            os.append(jnp.dot(p, vms, preferred_element_type=jnp.float32))
        o = jnp.concatenate(os, axis=0)
        a = lax.dot_general(o, wo_ref[l], _NT,
                            preferred_element_type=jnp.float32) + bo_ref[l]
        x = _ln(x + a, n1w_ref[l], n1b_ref[l])
        f = lax.dot_general(
            jax.nn.relu(
                lax.dot_general(x, w1_ref[l], _NT,
                                preferred_element_type=jnp.float32)
                + b1_ref[l]),
            w2_ref[l], _NT, preferred_element_type=jnp.float32) + b2_ref[l]
        x = _ln(x + f, n2w_ref[l], n2b_ref[l])
    out_ref[...] = x.reshape(BPS, L, D)


def kernel(X, timestamps, lengths, R_u, op1_vw, op1_vb, op1_sw, op1_sb,
           op2_vw, op2_vb, op2_sw, op2_sb, in_proj_w, in_proj_b,
           out_proj_w, out_proj_b, lin1_w, lin1_b, lin2_w, lin2_b,
           norm1_w, norm1_b, norm2_w, norm2_b):
    f32 = jnp.float32

    xt = X.transpose(0, 2, 1).reshape(B * F, L)
    rp = jnp.broadcast_to(R_u.reshape(F, D_OB)[:, None, :],
                          (F, L, D_OB)).reshape(F, C)
    times_rep = jnp.repeat(timestamps.transpose(1, 0), D_PE // 2, axis=1)
    ts_rep = jnp.tile(jnp.asarray(_TIMESCALES).reshape(1, D_PE // 2), (1, B))

    z, pe_sin, pe_cos = pl.pallas_call(
        _prop_pe_body,
        out_shape=[
            jax.ShapeDtypeStruct((B * F, C), f32),
            jax.ShapeDtypeStruct((L, B * (D_PE // 2)), f32),
            jax.ShapeDtypeStruct((L, B * (D_PE // 2)), f32),
        ],
    )(xt, rp,
      op1_vw, op1_vb.reshape(1, C), op1_sw, op1_sb.reshape(1, C),
      op2_vw, op2_vb.reshape(1, C), op2_sw, op2_sb.reshape(1, C),
      times_rep, ts_rep)

    out_units = z.reshape(B, F, L, D_OB).transpose(2, 0, 1, 3).reshape(L, B, D_MODEL)
    pe = jnp.concatenate([pe_sin.reshape(L, B, D_PE // 2),
                          pe_cos.reshape(L, B, D_PE // 2)], axis=-1)
    x0 = jnp.concatenate([out_units, pe], axis=2).transpose(1, 0, 2)

    mask = jnp.arange(L)[None, :] >= lengths
    neg = jnp.where(mask, jnp.float32(-1e30), jnp.float32(0.0))
    neg3 = neg.reshape(B, 1, L)

    wq = in_proj_w[:, 0 * D:1 * D, :]
    wk = in_proj_w[:, 1 * D:2 * D, :]
    wv = in_proj_w[:, 2 * D:3 * D, :]
    bq = in_proj_b[:, 0 * D:1 * D].reshape(N_LAYERS, 1, D)
    bk = in_proj_b[:, 1 * D:2 * D].reshape(N_LAYERS, 1, D)
    bv = in_proj_b[:, 2 * D:3 * D].reshape(N_LAYERS, 1, D)

    full = lambda shape: pl.BlockSpec(shape, lambda b: (0,) * len(shape))
    xout = pl.pallas_call(
        _tf_body,
        grid=(B // BPS,),
        in_specs=[
            pl.BlockSpec((BPS, L, D), lambda b: (b, 0, 0)),
            pl.BlockSpec((BPS, 1, L), lambda b: (b, 0, 0)),
            full((N_LAYERS, D, D)), full((N_LAYERS, D, D)), full((N_LAYERS, D, D)),
            full((N_LAYERS, 1, D)), full((N_LAYERS, 1, D)), full((N_LAYERS, 1, D)),
            full((N_LAYERS, D, D)), full((N_LAYERS, 1, D)),
            full((N_LAYERS, D_FFN, D)), full((N_LAYERS, 1, D_FFN)),
            full((N_LAYERS, D, D_FFN)), full((N_LAYERS, 1, D)),
            full((N_LAYERS, 1, D)), full((N_LAYERS, 1, D)),
            full((N_LAYERS, 1, D)), full((N_LAYERS, 1, D)),
        ],
        out_specs=pl.BlockSpec((BPS, L, D), lambda b: (b, 0, 0)),
        out_shape=jax.ShapeDtypeStruct((B, L, D), f32),
    )(x0, neg3, wq, wk, wv, bq, bk, bv,
      out_proj_w, out_proj_b.reshape(N_LAYERS, 1, D),
      lin1_w, lin1_b.reshape(N_LAYERS, 1, D_FFN),
      lin2_w, lin2_b.reshape(N_LAYERS, 1, D),
      norm1_w.reshape(N_LAYERS, 1, D), norm1_b.reshape(N_LAYERS, 1, D),
      norm2_w.reshape(N_LAYERS, 1, D), norm2_b.reshape(N_LAYERS, 1, D))

    return xout.transpose(1, 0, 2), mask
